# baseline (speedup 1.0000x reference)
import functools

import jax
import jax.numpy as jnp
from jax import lax
from jax.experimental import pallas as pl
from jax.experimental.pallas import tpu as pltpu
from jax.experimental.pallas import tpu_sc as plsc

SEQ = 200
B = 1024
V = 100000
D = 64
H = 64

NC = 2
NS = 16
NW = NC * NS
CHUNK = 128
PER_STEP = B // CHUNK
BASE_STEPS = SEQ // NW
EXTRA = SEQ - BASE_STEPS * NW


def _sc_gather(table, x128):
    mesh = plsc.VectorSubcoreMesh(core_axis_name="c", subcore_axis_name="s")

    @functools.partial(
        pl.kernel,
        out_type=jax.ShapeDtypeStruct((SEQ, B, D), jnp.float32),
        mesh=mesh,
        scratch_types=[
            pltpu.VMEM((PER_STEP, CHUNK), jnp.int32),
            pltpu.VMEM((B, D), jnp.float32),
            pltpu.SemaphoreType.DMA,
        ],
        compiler_params=pltpu.CompilerParams(use_tc_tiling_on_sc=False),
    )
    def k(table_hbm, idx_hbm, out_hbm, idx_v, rows_v, sem):
        wid = lax.axis_index("s") * NC + lax.axis_index("c")
        t0 = jnp.where(wid < NW - EXTRA,
                       BASE_STEPS * wid,
                       BASE_STEPS * wid + (wid - (NW - EXTRA)))
        t1 = t0 + jnp.where(wid < NW - EXTRA, BASE_STEPS, BASE_STEPS + 1)

        @pl.loop(t0, t1)
        def step(t):
            pltpu.sync_copy(idx_hbm.at[pl.ds(t * PER_STEP, PER_STEP)], idx_v)
            copies = [
                pltpu.async_copy(
                    table_hbm.at[idx_v.at[j]],
                    rows_v.at[pl.ds(j * CHUNK, CHUNK)],
                    sem,
                )
                for j in range(PER_STEP)
            ]
            for c in copies:
                c.wait()
            pltpu.sync_copy(rows_v, out_hbm.at[t])

    return k(table, x128)


T_BLK = 8
N_TBLK = SEQ // T_BLK


def _tc_gru(emb128, w_ih, w_hh, brz, bin_, bhn, interpret=False):
    rhs_t = (((1,), (1,)), ((), ()))

    def body(emb_ref, wih_ref, whh_ref, brz_ref, bin_ref, bhn_ref,
             out_ref, h_ref):
        t = pl.program_id(0)

        @pl.when(t == 0)
        def _():
            h_ref[...] = jnp.zeros_like(h_ref)

        wih = wih_ref[...]
        whh = whh_ref[...]
        for i in range(T_BLK):
            h = h_ref[...]
            p = emb_ref[i]
            gi_e = jax.lax.dot_general(
                wih, p[:, :D], rhs_t, preferred_element_type=jnp.float32)
            gi_o = jax.lax.dot_general(
                wih, p[:, D:], rhs_t, preferred_element_type=jnp.float32)
            gi = jnp.concatenate([gi_e, gi_o], axis=1)
            gh = jnp.dot(whh, h, preferred_element_type=jnp.float32)
            s = gi[: 2 * H] + gh[: 2 * H] + brz_ref[...]
            rz = 0.5 * jnp.tanh(0.5 * s) + 0.5
            r = rz[:H]
            z = rz[H:]
            n = jnp.tanh(gi[2 * H :] + bin_ref[...]
                         + r * (gh[2 * H :] + bhn_ref[...]))
            h_new = n + z * (h - n)
            h_ref[...] = h_new

        @pl.when(t == N_TBLK - 1)
        def _():
            out_ref[...] = h_ref[...]

    return pl.pallas_call(
        body,
        grid=(N_TBLK,),
        in_specs=[
            pl.BlockSpec((T_BLK, B // 2, 2 * D), lambda t: (t, 0, 0)),
            pl.BlockSpec((3 * H, D), lambda t: (0, 0)),
            pl.BlockSpec((3 * H, H), lambda t: (0, 0)),
            pl.BlockSpec((2 * H, B), lambda t: (0, 0)),
            pl.BlockSpec((H, B), lambda t: (0, 0)),
            pl.BlockSpec((H, B), lambda t: (0, 0)),
        ],
        out_specs=pl.BlockSpec((H, B), lambda t: (0, 0)),
        out_shape=jax.ShapeDtypeStruct((H, B), jnp.float32),
        scratch_shapes=[pltpu.VMEM((H, B), jnp.float32)],
        interpret=interpret,
    )(emb128, w_ih, w_hh, brz, bin_, bhn)


def kernel(x, table, W_ih, W_hh, b_ih, b_hh):
    x128 = x.astype(jnp.int32).reshape(SEQ * B // CHUNK, CHUNK)
    emb = _sc_gather(table, x128)
    brz = jnp.broadcast_to((b_ih[: 2 * H] + b_hh[: 2 * H])[:, None], (2 * H, B))
    bin_ = jnp.broadcast_to(b_ih[2 * H :][:, None], (H, B))
    bhn = jnp.broadcast_to(b_hh[2 * H :][:, None], (H, B))
    hn_t = _tc_gru(emb.reshape(SEQ, B // 2, 2 * D), W_ih, W_hh, brz, bin_, bhn)
    hn_e = hn_t[:, : B // 2].T
    hn_o = hn_t[:, B // 2 :].T
    return jnp.stack([hn_e, hn_o], axis=1).reshape(B, H)[None]


# speedup vs baseline: 1.9195x; 1.9195x over previous
import functools

import jax
import jax.numpy as jnp
from jax import lax
from jax.experimental import pallas as pl
from jax.experimental.pallas import tpu as pltpu
from jax.experimental.pallas import tpu_sc as plsc

SEQ = 200
B = 1024
V = 100000
D = 64
H = 64

NC = 2
NS = 16
NW = NC * NS
CHUNK = 128
PER_STEP = B // CHUNK
BASE_STEPS = SEQ // NW
EXTRA = SEQ - BASE_STEPS * NW


def _sc_gather(table, x128):
    mesh = plsc.VectorSubcoreMesh(core_axis_name="c", subcore_axis_name="s")

    @functools.partial(
        pl.kernel,
        out_type=jax.ShapeDtypeStruct((SEQ, B // 2, 2 * D), jnp.float32),
        mesh=mesh,
        scratch_types=[
            pltpu.VMEM((PER_STEP, CHUNK), jnp.int32),
            pltpu.VMEM((B, D), jnp.float32),
            pltpu.SemaphoreType.DMA,
        ],
        compiler_params=pltpu.CompilerParams(use_tc_tiling_on_sc=False),
    )
    def k(table_hbm, idx_hbm, out_hbm, idx_v, rows_v, sem):
        wid = lax.axis_index("s") * NC + lax.axis_index("c")
        t0 = jnp.where(wid < NW - EXTRA,
                       BASE_STEPS * wid,
                       BASE_STEPS * wid + (wid - (NW - EXTRA)))
        t1 = t0 + jnp.where(wid < NW - EXTRA, BASE_STEPS, BASE_STEPS + 1)

        @pl.loop(t0, t1)
        def step(t):
            pltpu.sync_copy(idx_hbm.at[pl.ds(t * PER_STEP, PER_STEP)], idx_v)
            copies = [
                pltpu.async_copy(
                    table_hbm.at[idx_v.at[j]],
                    rows_v.at[pl.ds(j * CHUNK, CHUNK)],
                    sem,
                )
                for j in range(PER_STEP)
            ]
            for c in copies:
                c.wait()
            pltpu.sync_copy(rows_v.at[pl.ds(0, B // 2)],
                            out_hbm.at[t, pl.ds(0, B // 2), pl.ds(0, D)])
            pltpu.sync_copy(rows_v.at[pl.ds(B // 2, B // 2)],
                            out_hbm.at[t, pl.ds(0, B // 2), pl.ds(D, D)])

    return k(table, x128)


T_BLK = 8
N_TBLK = SEQ // T_BLK


def _tc_gru(emb128, w_ih, w_hh, brz, bin_, bhn, interpret=False):
    rhs_t = (((1,), (1,)), ((), ()))

    def body(emb_ref, wih_ref, whh_ref, brz_ref, bin_ref, bhn_ref,
             out_ref, h_ref):
        t = pl.program_id(0)

        @pl.when(t == 0)
        def _():
            h_ref[...] = jnp.zeros_like(h_ref)

        wih = wih_ref[...]
        whh = whh_ref[...]
        for i in range(T_BLK):
            h = h_ref[...]
            p = emb_ref[i]
            gi_e = jax.lax.dot_general(
                wih, p[:, :D], rhs_t, preferred_element_type=jnp.float32)
            gi_o = jax.lax.dot_general(
                wih, p[:, D:], rhs_t, preferred_element_type=jnp.float32)
            gi = jnp.concatenate([gi_e, gi_o], axis=1)
            gh = jnp.dot(whh, h, preferred_element_type=jnp.float32)
            s = gi[: 2 * H] + gh[: 2 * H] + brz_ref[...]
            rz = 0.5 * jnp.tanh(0.5 * s) + 0.5
            r = rz[:H]
            z = rz[H:]
            n = jnp.tanh(gi[2 * H :] + bin_ref[...]
                         + r * (gh[2 * H :] + bhn_ref[...]))
            h_new = n + z * (h - n)
            h_ref[...] = h_new

        @pl.when(t == N_TBLK - 1)
        def _():
            out_ref[...] = h_ref[...]

    return pl.pallas_call(
        body,
        grid=(N_TBLK,),
        in_specs=[
            pl.BlockSpec((T_BLK, B // 2, 2 * D), lambda t: (t, 0, 0)),
            pl.BlockSpec((3 * H, D), lambda t: (0, 0)),
            pl.BlockSpec((3 * H, H), lambda t: (0, 0)),
            pl.BlockSpec((2 * H, B), lambda t: (0, 0)),
            pl.BlockSpec((H, B), lambda t: (0, 0)),
            pl.BlockSpec((H, B), lambda t: (0, 0)),
        ],
        out_specs=pl.BlockSpec((H, B), lambda t: (0, 0)),
        out_shape=jax.ShapeDtypeStruct((H, B), jnp.float32),
        scratch_shapes=[pltpu.VMEM((H, B), jnp.float32)],
        interpret=interpret,
    )(emb128, w_ih, w_hh, brz, bin_, bhn)


def kernel(x, table, W_ih, W_hh, b_ih, b_hh):
    xi = x.astype(jnp.int32)
    x_perm = jnp.concatenate([xi[:, 0::2], xi[:, 1::2]], axis=1)
    emb = _sc_gather(table, x_perm.reshape(SEQ * B // CHUNK, CHUNK))
    brz = jnp.broadcast_to((b_ih[: 2 * H] + b_hh[: 2 * H])[:, None], (2 * H, B))
    bin_ = jnp.broadcast_to(b_ih[2 * H :][:, None], (H, B))
    bhn = jnp.broadcast_to(b_hh[2 * H :][:, None], (H, B))
    hn_t = _tc_gru(emb, W_ih, W_hh, brz, bin_, bhn)
    hn_e = hn_t[:, : B // 2].T
    hn_o = hn_t[:, B // 2 :].T
    return jnp.stack([hn_e, hn_o], axis=1).reshape(B, H)[None]


# speedup vs baseline: 2.0404x; 1.0630x over previous
import functools

import jax
import jax.numpy as jnp
from jax import lax
from jax.experimental import pallas as pl
from jax.experimental.pallas import tpu as pltpu
from jax.experimental.pallas import tpu_sc as plsc

SEQ = 200
B = 1024
V = 100000
D = 64
H = 64

NC = 2
NS = 16
NW = NC * NS
CHUNK = 128
PER_STEP = B // CHUNK
BASE_STEPS = SEQ // NW
EXTRA = SEQ - BASE_STEPS * NW


def _sc_gather(table, x128):
    mesh = plsc.VectorSubcoreMesh(core_axis_name="c", subcore_axis_name="s")

    @functools.partial(
        pl.kernel,
        out_type=jax.ShapeDtypeStruct((SEQ, B // 2, 2 * D), jnp.float32),
        mesh=mesh,
        scratch_types=[
            pltpu.VMEM((PER_STEP, CHUNK), jnp.int32),
            pltpu.VMEM((B, D), jnp.float32),
            pltpu.SemaphoreType.DMA,
        ],
        compiler_params=pltpu.CompilerParams(use_tc_tiling_on_sc=False),
    )
    def k(table_hbm, idx_hbm, out_hbm, idx_v, rows_v, sem):
        wid = lax.axis_index("s") * NC + lax.axis_index("c")
        t0 = jnp.where(wid < NW - EXTRA,
                       BASE_STEPS * wid,
                       BASE_STEPS * wid + (wid - (NW - EXTRA)))
        t1 = t0 + jnp.where(wid < NW - EXTRA, BASE_STEPS, BASE_STEPS + 1)

        @pl.loop(t0, t1)
        def step(t):
            pltpu.sync_copy(idx_hbm.at[pl.ds(t * PER_STEP, PER_STEP)], idx_v)
            copies = [
                pltpu.async_copy(
                    table_hbm.at[idx_v.at[j]],
                    rows_v.at[pl.ds(j * CHUNK, CHUNK)],
                    sem,
                )
                for j in range(PER_STEP)
            ]
            for c in copies:
                c.wait()
            pltpu.sync_copy(rows_v.at[pl.ds(0, B // 2)],
                            out_hbm.at[t, pl.ds(0, B // 2), pl.ds(0, D)])
            pltpu.sync_copy(rows_v.at[pl.ds(B // 2, B // 2)],
                            out_hbm.at[t, pl.ds(0, B // 2), pl.ds(D, D)])

    return k(table, x128)


T_BLK = 8
N_TBLK = SEQ // T_BLK


def _tc_gru(emb128, w_ih, w_hh, brz, bin_, bhn, interpret=False):
    rhs_t = (((1,), (1,)), ((), ()))

    def body(emb_ref, wih_ref, whh_ref, brz_ref, bin_ref, bhn_ref,
             out_ref, h_ref):
        t = pl.program_id(0)

        @pl.when(t == 0)
        def _():
            h_ref[...] = jnp.zeros_like(h_ref)

        wih = wih_ref[...]
        whh = whh_ref[...]
        for i in range(T_BLK):
            h = h_ref[...]
            p = emb_ref[i]
            gi_lo = jax.lax.dot_general(
                wih, p[:, :D], rhs_t, preferred_element_type=jnp.float32)
            gi_hi = jax.lax.dot_general(
                wih, p[:, D:], rhs_t, preferred_element_type=jnp.float32)
            gi = jnp.concatenate([gi_lo, gi_hi], axis=1)
            gh = jnp.dot(whh, h, preferred_element_type=jnp.float32)
            s = gi[: 2 * H] + gh[: 2 * H] + brz_ref[...]
            rz = 0.5 * jnp.tanh(0.5 * s) + 0.5
            r = rz[:H]
            z = rz[H:]
            n = jnp.tanh(gi[2 * H :] + bin_ref[...]
                         + r * (gh[2 * H :] + bhn_ref[...]))
            h_new = n + z * (h - n)
            h_ref[...] = h_new

        @pl.when(t == N_TBLK - 1)
        def _():
            out_ref[...] = h_ref[...]

    return pl.pallas_call(
        body,
        grid=(N_TBLK,),
        in_specs=[
            pl.BlockSpec((T_BLK, B // 2, 2 * D), lambda t: (t, 0, 0)),
            pl.BlockSpec((3 * H, D), lambda t: (0, 0)),
            pl.BlockSpec((3 * H, H), lambda t: (0, 0)),
            pl.BlockSpec((2 * H, B), lambda t: (0, 0)),
            pl.BlockSpec((H, B), lambda t: (0, 0)),
            pl.BlockSpec((H, B), lambda t: (0, 0)),
        ],
        out_specs=pl.BlockSpec((H, B), lambda t: (0, 0)),
        out_shape=jax.ShapeDtypeStruct((H, B), jnp.float32),
        scratch_shapes=[pltpu.VMEM((H, B), jnp.float32)],
        interpret=interpret,
    )(emb128, w_ih, w_hh, brz, bin_, bhn)


def kernel(x, table, W_ih, W_hh, b_ih, b_hh):
    x128 = x.astype(jnp.int32).reshape(SEQ * B // CHUNK, CHUNK)
    emb = _sc_gather(table, x128)
    brz = jnp.broadcast_to((b_ih[: 2 * H] + b_hh[: 2 * H])[:, None], (2 * H, B))
    bin_ = jnp.broadcast_to(b_ih[2 * H :][:, None], (H, B))
    bhn = jnp.broadcast_to(b_hh[2 * H :][:, None], (H, B))
    hn_t = _tc_gru(emb, W_ih, W_hh, brz, bin_, bhn)
    return hn_t.T[None]


# speedup vs baseline: 2.0428x; 1.0012x over previous
import functools

import jax
import jax.numpy as jnp
from jax import lax
from jax.experimental import pallas as pl
from jax.experimental.pallas import tpu as pltpu
from jax.experimental.pallas import tpu_sc as plsc

SEQ = 200
B = 1024
V = 100000
D = 64
H = 64

NC = 2
NS = 16
NW = NC * NS
CHUNK = 128
PER_STEP = B // CHUNK
BASE_STEPS = SEQ // NW
EXTRA = SEQ - BASE_STEPS * NW


def _sc_gather(table, x128):
    mesh = plsc.VectorSubcoreMesh(core_axis_name="c", subcore_axis_name="s")

    @functools.partial(
        pl.kernel,
        out_type=jax.ShapeDtypeStruct((SEQ, B // 2, 2 * D), jnp.float32),
        mesh=mesh,
        scratch_types=[
            pltpu.VMEM((PER_STEP, CHUNK), jnp.int32),
            pltpu.VMEM((B, D), jnp.float32),
            pltpu.SemaphoreType.DMA,
        ],
        compiler_params=pltpu.CompilerParams(use_tc_tiling_on_sc=False),
    )
    def k(table_hbm, idx_hbm, out_hbm, idx_v, rows_v, sem):
        wid = lax.axis_index("s") * NC + lax.axis_index("c")
        t0 = jnp.where(wid < NW - EXTRA,
                       BASE_STEPS * wid,
                       BASE_STEPS * wid + (wid - (NW - EXTRA)))
        t1 = t0 + jnp.where(wid < NW - EXTRA, BASE_STEPS, BASE_STEPS + 1)

        @pl.loop(t0, t1)
        def step(t):
            pltpu.sync_copy(idx_hbm.at[t // 8, :, t % 8], idx_v)
            copies = [
                pltpu.async_copy(
                    table_hbm.at[idx_v.at[j]],
                    rows_v.at[pl.ds(j * CHUNK, CHUNK)],
                    sem,
                )
                for j in range(PER_STEP)
            ]
            for c in copies:
                c.wait()
            pltpu.sync_copy(rows_v.at[pl.ds(0, B // 2)],
                            out_hbm.at[t, pl.ds(0, B // 2), pl.ds(0, D)])
            pltpu.sync_copy(rows_v.at[pl.ds(B // 2, B // 2)],
                            out_hbm.at[t, pl.ds(0, B // 2), pl.ds(D, D)])

    return k(table, x128)


T_BLK = 8
N_TBLK = SEQ // T_BLK


def _tc_gru(emb128, w_ih, w_hh, brz, bin_, bhn, interpret=False):
    rhs_t = (((1,), (1,)), ((), ()))

    def body(emb_ref, wih_ref, whh_ref, brz_ref, bin_ref, bhn_ref,
             out_ref, h_ref):
        t = pl.program_id(0)

        @pl.when(t == 0)
        def _():
            h_ref[...] = jnp.zeros_like(h_ref)

        wih = wih_ref[...]
        whh = whh_ref[...]
        for i in range(T_BLK):
            h = h_ref[...]
            p = emb_ref[i]
            gi_lo = jax.lax.dot_general(
                wih, p[:, :D], rhs_t, preferred_element_type=jnp.float32)
            gi_hi = jax.lax.dot_general(
                wih, p[:, D:], rhs_t, preferred_element_type=jnp.float32)
            gi = jnp.concatenate([gi_lo, gi_hi], axis=1)
            gh = jnp.dot(whh, h, preferred_element_type=jnp.float32)
            s = gi[: 2 * H] + gh[: 2 * H] + brz_ref[...]
            rz = 0.5 * jnp.tanh(0.5 * s) + 0.5
            r = rz[:H]
            z = rz[H:]
            n = jnp.tanh(gi[2 * H :] + bin_ref[...]
                         + r * (gh[2 * H :] + bhn_ref[...]))
            h_new = n + z * (h - n)
            h_ref[...] = h_new

        @pl.when(t == N_TBLK - 1)
        def _():
            out_ref[...] = h_ref[...]

    return pl.pallas_call(
        body,
        grid=(N_TBLK,),
        in_specs=[
            pl.BlockSpec((T_BLK, B // 2, 2 * D), lambda t: (t, 0, 0)),
            pl.BlockSpec((3 * H, D), lambda t: (0, 0)),
            pl.BlockSpec((3 * H, H), lambda t: (0, 0)),
            pl.BlockSpec((2 * H, B), lambda t: (0, 0)),
            pl.BlockSpec((H, B), lambda t: (0, 0)),
            pl.BlockSpec((H, B), lambda t: (0, 0)),
        ],
        out_specs=pl.BlockSpec((H, B), lambda t: (0, 0)),
        out_shape=jax.ShapeDtypeStruct((H, B), jnp.float32),
        scratch_shapes=[pltpu.VMEM((H, B), jnp.float32)],
        interpret=interpret,
    )(emb128, w_ih, w_hh, brz, bin_, bhn)


def kernel(x, table, W_ih, W_hh, b_ih, b_hh):
    x4 = x.astype(jnp.int32).reshape(SEQ // 8, 8, B // CHUNK, CHUNK)
    x4 = x4.transpose(0, 2, 1, 3)
    emb = _sc_gather(table, x4)
    brz = jnp.broadcast_to((b_ih[: 2 * H] + b_hh[: 2 * H])[:, None], (2 * H, B))
    bin_ = jnp.broadcast_to(b_ih[2 * H :][:, None], (H, B))
    bhn = jnp.broadcast_to(b_hh[2 * H :][:, None], (H, B))
    hn_t = _tc_gru(emb, W_ih, W_hh, brz, bin_, bhn)
    return hn_t.T[None]


# speedup vs baseline: 2.2131x; 1.0834x over previous
import functools

import jax
import jax.numpy as jnp
from jax import lax
from jax.experimental import pallas as pl
from jax.experimental.pallas import tpu as pltpu
from jax.experimental.pallas import tpu_sc as plsc

SEQ = 200
B = 1024
V = 100000
D = 64
H = 64

NC = 2
NS = 16
NW = NC * NS
CHUNK = 128
PER_STEP = B // CHUNK

NCH = 2
CH_STEPS = SEQ // NCH
CBASE = CH_STEPS // NW
CEXTRA = CH_STEPS - CBASE * NW


def _sc_gather(table, x4, c0):
    mesh = plsc.VectorSubcoreMesh(core_axis_name="c", subcore_axis_name="s")

    @functools.partial(
        pl.kernel,
        out_type=jax.ShapeDtypeStruct((CH_STEPS, B // 2, 2 * D), jnp.float32),
        mesh=mesh,
        scratch_types=[
            pltpu.VMEM((PER_STEP, CHUNK), jnp.int32),
            pltpu.VMEM((B, D), jnp.float32),
            pltpu.SemaphoreType.DMA,
        ],
        compiler_params=pltpu.CompilerParams(use_tc_tiling_on_sc=False),
    )
    def k(table_hbm, idx_hbm, out_hbm, idx_v, rows_v, sem):
        wid = lax.axis_index("s") * NC + lax.axis_index("c")
        t0 = jnp.where(wid < NW - CEXTRA,
                       CBASE * wid,
                       CBASE * wid + (wid - (NW - CEXTRA)))
        t1 = t0 + jnp.where(wid < NW - CEXTRA, CBASE, CBASE + 1)

        @pl.loop(t0, t1)
        def step(t):
            ta = t + c0
            pltpu.sync_copy(idx_hbm.at[ta // 8, :, ta % 8], idx_v)
            copies = [
                pltpu.async_copy(
                    table_hbm.at[idx_v.at[j]],
                    rows_v.at[pl.ds(j * CHUNK, CHUNK)],
                    sem,
                )
                for j in range(PER_STEP)
            ]
            for c in copies:
                c.wait()
            pltpu.sync_copy(rows_v.at[pl.ds(0, B // 2)],
                            out_hbm.at[t, pl.ds(0, B // 2), pl.ds(0, D)])
            pltpu.sync_copy(rows_v.at[pl.ds(B // 2, B // 2)],
                            out_hbm.at[t, pl.ds(0, B // 2), pl.ds(D, D)])

    return k(table, x4)


T_BLK = 10
N_TBLK = CH_STEPS // T_BLK


def _tc_gru(emb128, h_in, w_ih, w_hh, brz, bin_, bhn, interpret=False):
    rhs_t = (((1,), (1,)), ((), ()))

    def body(emb_ref, hin_ref, wih_ref, whh_ref, brz_ref, bin_ref, bhn_ref,
             out_ref, h_ref):
        t = pl.program_id(0)

        @pl.when(t == 0)
        def _():
            h_ref[...] = hin_ref[...]

        wih = wih_ref[...]
        whh = whh_ref[...]
        for i in range(T_BLK):
            h = h_ref[...]
            p = emb_ref[i]
            gi_lo = jax.lax.dot_general(
                wih, p[:, :D], rhs_t, preferred_element_type=jnp.float32)
            gi_hi = jax.lax.dot_general(
                wih, p[:, D:], rhs_t, preferred_element_type=jnp.float32)
            gi = jnp.concatenate([gi_lo, gi_hi], axis=1)
            gh = jnp.dot(whh, h, preferred_element_type=jnp.float32)
            s = gi[: 2 * H] + gh[: 2 * H] + brz_ref[...]
            rz = 0.5 * jnp.tanh(0.5 * s) + 0.5
            r = rz[:H]
            z = rz[H:]
            n = jnp.tanh(gi[2 * H :] + bin_ref[...]
                         + r * (gh[2 * H :] + bhn_ref[...]))
            h_new = n + z * (h - n)
            h_ref[...] = h_new

        @pl.when(t == N_TBLK - 1)
        def _():
            out_ref[...] = h_ref[...]

    return pl.pallas_call(
        body,
        grid=(N_TBLK,),
        in_specs=[
            pl.BlockSpec((T_BLK, B // 2, 2 * D), lambda t: (t, 0, 0)),
            pl.BlockSpec((H, B), lambda t: (0, 0)),
            pl.BlockSpec((3 * H, D), lambda t: (0, 0)),
            pl.BlockSpec((3 * H, H), lambda t: (0, 0)),
            pl.BlockSpec((2 * H, B), lambda t: (0, 0)),
            pl.BlockSpec((H, B), lambda t: (0, 0)),
            pl.BlockSpec((H, B), lambda t: (0, 0)),
        ],
        out_specs=pl.BlockSpec((H, B), lambda t: (0, 0)),
        out_shape=jax.ShapeDtypeStruct((H, B), jnp.float32),
        scratch_shapes=[pltpu.VMEM((H, B), jnp.float32)],
        interpret=interpret,
    )(emb128, h_in, w_ih, w_hh, brz, bin_, bhn)


def kernel(x, table, W_ih, W_hh, b_ih, b_hh):
    x4 = x.astype(jnp.int32).reshape(SEQ // 8, 8, B // CHUNK, CHUNK)
    x4 = x4.transpose(0, 2, 1, 3)
    embs = [_sc_gather(table, x4, c * CH_STEPS) for c in range(NCH)]
    brz = jnp.broadcast_to((b_ih[: 2 * H] + b_hh[: 2 * H])[:, None], (2 * H, B))
    bin_ = jnp.broadcast_to(b_ih[2 * H :][:, None], (H, B))
    bhn = jnp.broadcast_to(b_hh[2 * H :][:, None], (H, B))
    h = jnp.zeros((H, B), jnp.float32)
    for c in range(NCH):
        h = _tc_gru(embs[c], h, W_ih, W_hh, brz, bin_, bhn)
    return h.T[None]


# speedup vs baseline: 2.2643x; 1.0231x over previous
import functools

import jax
import jax.numpy as jnp
from jax import lax
from jax.experimental import pallas as pl
from jax.experimental.pallas import tpu as pltpu
from jax.experimental.pallas import tpu_sc as plsc

SEQ = 200
B = 1024
V = 100000
D = 64
H = 64

NC = 2
NS = 16
NW = NC * NS
CHUNK = 128
PER_STEP = B // CHUNK

NCH = 4
CH_STEPS = SEQ // NCH
CBASE = CH_STEPS // NW
CEXTRA = CH_STEPS - CBASE * NW


def _sc_gather(table, x4, c0):
    mesh = plsc.VectorSubcoreMesh(core_axis_name="c", subcore_axis_name="s")

    @functools.partial(
        pl.kernel,
        out_type=jax.ShapeDtypeStruct((CH_STEPS, B // 2, 2 * D), jnp.float32),
        mesh=mesh,
        scratch_types=[
            pltpu.VMEM((PER_STEP, CHUNK), jnp.int32),
            pltpu.VMEM((B, D), jnp.float32),
            pltpu.SemaphoreType.DMA,
        ],
        compiler_params=pltpu.CompilerParams(use_tc_tiling_on_sc=False),
    )
    def k(table_hbm, idx_hbm, out_hbm, idx_v, rows_v, sem):
        wid = lax.axis_index("s") * NC + lax.axis_index("c")
        t0 = jnp.where(wid < NW - CEXTRA,
                       CBASE * wid,
                       CBASE * wid + (wid - (NW - CEXTRA)))
        t1 = t0 + jnp.where(wid < NW - CEXTRA, CBASE, CBASE + 1)

        @pl.loop(t0, t1)
        def step(t):
            ta = t + c0
            pltpu.sync_copy(idx_hbm.at[ta // 8, :, ta % 8], idx_v)
            copies = [
                pltpu.async_copy(
                    table_hbm.at[idx_v.at[j]],
                    rows_v.at[pl.ds(j * CHUNK, CHUNK)],
                    sem,
                )
                for j in range(PER_STEP)
            ]
            for c in copies:
                c.wait()
            pltpu.sync_copy(rows_v.at[pl.ds(0, B // 2)],
                            out_hbm.at[t, pl.ds(0, B // 2), pl.ds(0, D)])
            pltpu.sync_copy(rows_v.at[pl.ds(B // 2, B // 2)],
                            out_hbm.at[t, pl.ds(0, B // 2), pl.ds(D, D)])

    return k(table, x4)


T_BLK = 10
N_TBLK = CH_STEPS // T_BLK


def _tc_gru(emb128, h_in, w_ih, w_hh, brz, bin_, bhn, interpret=False):
    rhs_t = (((1,), (1,)), ((), ()))

    def body(emb_ref, hin_ref, wih_ref, whh_ref, brz_ref, bin_ref, bhn_ref,
             out_ref, h_ref):
        t = pl.program_id(0)

        @pl.when(t == 0)
        def _():
            h_ref[...] = hin_ref[...]

        wih = wih_ref[...]
        whh = whh_ref[...]
        for i in range(T_BLK):
            h = h_ref[...]
            p = emb_ref[i]
            gi_lo = jax.lax.dot_general(
                wih, p[:, :D], rhs_t, preferred_element_type=jnp.float32)
            gi_hi = jax.lax.dot_general(
                wih, p[:, D:], rhs_t, preferred_element_type=jnp.float32)
            gi = jnp.concatenate([gi_lo, gi_hi], axis=1)
            gh = jnp.dot(whh, h, preferred_element_type=jnp.float32)
            s = gi[: 2 * H] + gh[: 2 * H] + brz_ref[...]
            rz = 0.5 * jnp.tanh(0.5 * s) + 0.5
            r = rz[:H]
            z = rz[H:]
            n = jnp.tanh(gi[2 * H :] + bin_ref[...]
                         + r * (gh[2 * H :] + bhn_ref[...]))
            h_new = n + z * (h - n)
            h_ref[...] = h_new

        @pl.when(t == N_TBLK - 1)
        def _():
            out_ref[...] = h_ref[...]

    return pl.pallas_call(
        body,
        grid=(N_TBLK,),
        in_specs=[
            pl.BlockSpec((T_BLK, B // 2, 2 * D), lambda t: (t, 0, 0)),
            pl.BlockSpec((H, B), lambda t: (0, 0)),
            pl.BlockSpec((3 * H, D), lambda t: (0, 0)),
            pl.BlockSpec((3 * H, H), lambda t: (0, 0)),
            pl.BlockSpec((2 * H, B), lambda t: (0, 0)),
            pl.BlockSpec((H, B), lambda t: (0, 0)),
            pl.BlockSpec((H, B), lambda t: (0, 0)),
        ],
        out_specs=pl.BlockSpec((H, B), lambda t: (0, 0)),
        out_shape=jax.ShapeDtypeStruct((H, B), jnp.float32),
        scratch_shapes=[pltpu.VMEM((H, B), jnp.float32)],
        interpret=interpret,
    )(emb128, h_in, w_ih, w_hh, brz, bin_, bhn)


def kernel(x, table, W_ih, W_hh, b_ih, b_hh):
    x4 = x.astype(jnp.int32).reshape(SEQ // 8, 8, B // CHUNK, CHUNK)
    x4 = x4.transpose(0, 2, 1, 3)
    embs = [_sc_gather(table, x4, c * CH_STEPS) for c in range(NCH)]
    brz = jnp.broadcast_to((b_ih[: 2 * H] + b_hh[: 2 * H])[:, None], (2 * H, B))
    bin_ = jnp.broadcast_to(b_ih[2 * H :][:, None], (H, B))
    bhn = jnp.broadcast_to(b_hh[2 * H :][:, None], (H, B))
    h = jnp.zeros((H, B), jnp.float32)
    for c in range(NCH):
        h = _tc_gru(embs[c], h, W_ih, W_hh, brz, bin_, bhn)
    return h.T[None]


# speedup vs baseline: 2.5574x; 1.1295x over previous
import functools

import jax
import jax.numpy as jnp
from jax import lax
from jax.experimental import pallas as pl
from jax.experimental.pallas import tpu as pltpu
from jax.experimental.pallas import tpu_sc as plsc

SEQ = 200
B = 1024
V = 100000
D = 64
H = 64

NC = 2
NS = 16
NW = NC * NS
CHUNK = 128
PER_STEP = B // CHUNK

NCH = 4
CH_STEPS = SEQ // NCH
CBASE = CH_STEPS // NW
CEXTRA = CH_STEPS - CBASE * NW


BKL = 8192
NBK = (V + BKL - 1) // BKL


def _tc_repack(tbl_t, interpret=False):
    def body(in_ref, out_ref):
        t = in_ref[...].T
        t3 = t.reshape(BKL // 2, 2, D)
        out_ref[...] = jnp.concatenate([t3[:, 0, :], t3[:, 1, :]], axis=1)

    return pl.pallas_call(
        body,
        grid=(NBK,),
        in_specs=[pl.BlockSpec((D, BKL), lambda t: (0, t))],
        out_specs=pl.BlockSpec((BKL // 2, 2 * D), lambda t: (t, 0)),
        out_shape=jax.ShapeDtypeStruct((V // 2, 2 * D), jnp.float32),
        interpret=interpret,
    )(tbl_t)


def _sc_gather(table, x4, c0):
    mesh = plsc.VectorSubcoreMesh(core_axis_name="c", subcore_axis_name="s")

    @functools.partial(
        pl.kernel,
        out_type=jax.ShapeDtypeStruct((CH_STEPS, B // 2, 2 * D), jnp.float32),
        mesh=mesh,
        scratch_types=[
            pltpu.VMEM((PER_STEP, CHUNK), jnp.int32),
            pltpu.VMEM((B, D), jnp.float32),
            pltpu.SemaphoreType.DMA,
        ],
        compiler_params=pltpu.CompilerParams(use_tc_tiling_on_sc=False),
    )
    def k(table_hbm, idx_hbm, out_hbm, idx_v, rows_v, sem):
        wid = lax.axis_index("s") * NC + lax.axis_index("c")
        t0 = jnp.where(wid < NW - CEXTRA,
                       CBASE * wid,
                       CBASE * wid + (wid - (NW - CEXTRA)))
        t1 = t0 + jnp.where(wid < NW - CEXTRA, CBASE, CBASE + 1)

        @pl.loop(t0, t1)
        def step(t):
            ta = t + c0
            pltpu.sync_copy(idx_hbm.at[ta // 8, :, ta % 8], idx_v)
            copies = [
                pltpu.async_copy(
                    table_hbm.at[idx_v.at[j]],
                    rows_v.at[pl.ds(j * CHUNK, CHUNK)],
                    sem,
                )
                for j in range(PER_STEP)
            ]
            for c in copies:
                c.wait()
            pltpu.sync_copy(rows_v.at[pl.ds(0, B // 2)],
                            out_hbm.at[t, pl.ds(0, B // 2), pl.ds(0, D)])
            pltpu.sync_copy(rows_v.at[pl.ds(B // 2, B // 2)],
                            out_hbm.at[t, pl.ds(0, B // 2), pl.ds(D, D)])

    return k(table, x4)


T_BLK = 10
N_TBLK = CH_STEPS // T_BLK


def _tc_gru(emb128, h_in, w_ih, w_hh, brz, bin_, bhn, interpret=False):
    rhs_t = (((1,), (1,)), ((), ()))

    def body(emb_ref, hin_ref, wih_ref, whh_ref, brz_ref, bin_ref, bhn_ref,
             out_ref, h_ref):
        t = pl.program_id(0)

        @pl.when(t == 0)
        def _():
            h_ref[...] = hin_ref[...]

        wih = wih_ref[...]
        whh = whh_ref[...]
        for i in range(T_BLK):
            h = h_ref[...]
            p = emb_ref[i]
            gi_lo = jax.lax.dot_general(
                wih, p[:, :D], rhs_t, preferred_element_type=jnp.float32)
            gi_hi = jax.lax.dot_general(
                wih, p[:, D:], rhs_t, preferred_element_type=jnp.float32)
            gi = jnp.concatenate([gi_lo, gi_hi], axis=1)
            gh = jnp.dot(whh, h, preferred_element_type=jnp.float32)
            s = gi[: 2 * H] + gh[: 2 * H] + brz_ref[...]
            rz = 0.5 * jnp.tanh(0.5 * s) + 0.5
            r = rz[:H]
            z = rz[H:]
            n = jnp.tanh(gi[2 * H :] + bin_ref[...]
                         + r * (gh[2 * H :] + bhn_ref[...]))
            h_new = n + z * (h - n)
            h_ref[...] = h_new

        @pl.when(t == N_TBLK - 1)
        def _():
            out_ref[...] = h_ref[...]

    return pl.pallas_call(
        body,
        grid=(N_TBLK,),
        in_specs=[
            pl.BlockSpec((T_BLK, B // 2, 2 * D), lambda t: (t, 0, 0)),
            pl.BlockSpec((H, B), lambda t: (0, 0)),
            pl.BlockSpec((3 * H, D), lambda t: (0, 0)),
            pl.BlockSpec((3 * H, H), lambda t: (0, 0)),
            pl.BlockSpec((2 * H, B), lambda t: (0, 0)),
            pl.BlockSpec((H, B), lambda t: (0, 0)),
            pl.BlockSpec((H, B), lambda t: (0, 0)),
        ],
        out_specs=pl.BlockSpec((H, B), lambda t: (0, 0)),
        out_shape=jax.ShapeDtypeStruct((H, B), jnp.float32),
        scratch_shapes=[pltpu.VMEM((H, B), jnp.float32)],
        interpret=interpret,
    )(emb128, h_in, w_ih, w_hh, brz, bin_, bhn)


def kernel(x, table, W_ih, W_hh, b_ih, b_hh):
    x4 = x.astype(jnp.int32).reshape(SEQ // 8, 8, B // CHUNK, CHUNK)
    x4 = x4.transpose(0, 2, 1, 3)
    tbl_t = lax.optimization_barrier(table.T)
    lin_table = _tc_repack(tbl_t).reshape(V, D)
    embs = [_sc_gather(lin_table, x4, c * CH_STEPS) for c in range(NCH)]
    brz = jnp.broadcast_to((b_ih[: 2 * H] + b_hh[: 2 * H])[:, None], (2 * H, B))
    bin_ = jnp.broadcast_to(b_ih[2 * H :][:, None], (H, B))
    bhn = jnp.broadcast_to(b_hh[2 * H :][:, None], (H, B))
    h = jnp.zeros((H, B), jnp.float32)
    for c in range(NCH):
        h = _tc_gru(embs[c], h, W_ih, W_hh, brz, bin_, bhn)
    return h.T[None]
